# baseline (speedup 1.0000x reference)
import jax
import jax.numpy as jnp
from jax import lax
from jax.experimental import pallas as pl
from jax.experimental.pallas import tpu as pltpu
from jax.experimental.pallas import tpu_sc as plsc

D = 64
P = 20
N = 50
B = 4096
NC, NS = 2, 16
NW = NC * NS
BPW = B // NW
PP = 24
NN = 56
EP = 16
EN = 8
LANES = 16
KD = D // LANES


def _prep_body(p_idx, n_idx, p_out, n_out, sp, sn, fp, fn):
    wid = lax.axis_index("s") * NC + lax.axis_index("c")
    base_b = wid * BPW
    pltpu.sync_copy(p_idx.at[pl.ds(base_b, BPW)], sp)
    pltpu.sync_copy(n_idx.at[pl.ds(base_b, BPW)], sn)
    lane = lax.iota(jnp.int32, LANES)
    zero = jnp.zeros((LANES,), jnp.int32)

    @pl.loop(0, BPW)
    def _row(r):
        plsc.store_scatter(fp, [r * PP + (PP - LANES) + lane], zero)
        for c in (0, P - LANES):
            plsc.store_scatter(fp, [r * PP + c + lane],
                               sp[r, pl.ds(c, LANES)])
        plsc.store_scatter(fn, [r * NN + (NN - LANES) + lane], zero)
        for c in (0, LANES, 2 * LANES, N - LANES):
            plsc.store_scatter(fn, [r * NN + c + lane],
                               sn[r, pl.ds(c, LANES)])

    pltpu.sync_copy(fp, p_out.at[pl.ds(base_b * PP, BPW * PP)])
    pltpu.sync_copy(fn, n_out.at[pl.ds(base_b * NN, BPW * NN)])


_prep = pl.kernel(
    _prep_body,
    out_type=(jax.ShapeDtypeStruct((B * PP,), jnp.int32),
              jax.ShapeDtypeStruct((B * NN,), jnp.int32)),
    mesh=plsc.VectorSubcoreMesh(core_axis_name="c", subcore_axis_name="s",
                                num_cores=NC, num_subcores=NS),
    scratch_types=[
        pltpu.VMEM((BPW, P), jnp.int32),
        pltpu.VMEM((BPW, N), jnp.int32),
        pltpu.VMEM((BPW * PP,), jnp.int32),
        pltpu.VMEM((BPW * NN,), jnp.int32),
    ],
    compiler_params=pltpu.CompilerParams(needs_layout_passes=False),
)


def _sc_body(table, u_idx, p_lin, n_lin, sc_out, nsc_out,
             idx_u, idx_p, idx_n, u_rows, ring, s_v, n_v,
             sem_u, sem_g0, sem_g1):
    wid = lax.axis_index("s") * NC + lax.axis_index("c")
    base_b = wid * BPW
    pltpu.sync_copy(u_idx.at[pl.ds(base_b, BPW)], idx_u)
    pltpu.sync_copy(p_lin.at[pl.ds(base_b * PP, BPW * PP)], idx_p)
    pltpu.sync_copy(n_lin.at[pl.ds(base_b * NN, BPW * NN)], idx_n)
    pltpu.async_copy(table.at[idx_u], u_rows, sem_u).wait()
    sems = (sem_g0, sem_g1)

    def _dot_u(buf, row, r0, stride):
        acc = [jnp.zeros((LANES,), jnp.float32) for _ in range(KD)]
        for j in range(stride):
            r = r0 + j
            for k in range(KD):
                acc[k] = acc[k] + buf[r, pl.ds(LANES * k, LANES)]
        dp = acc[0] * u_rows[row, pl.ds(0, LANES)]
        for k in range(1, KD):
            dp = dp + acc[k] * u_rows[row, pl.ds(LANES * k, LANES)]
        return dp

    def _pipelined_pass(idx_1d, stride_e, real_e, epc, out_v):
        nch = BPW // epc

        def fire(c, par):
            for j in range(epc):
                pltpu.async_copy(
                    table.at[idx_1d.at[pl.ds((c * epc + j) * stride_e,
                                             stride_e)]],
                    ring.at[par, pl.ds(stride_e * j, stride_e)], sems[par])

        def wait(c, par):
            for j in range(epc):
                pltpu.make_async_copy(
                    table.at[idx_1d.at[pl.ds((c * epc + j) * stride_e,
                                             stride_e)]],
                    ring.at[par, pl.ds(stride_e * j, stride_e)],
                    sems[par]).wait()

        def compute(c, par):
            @pl.loop(0, epc)
            def _elem(e):
                row = c * epc + e
                out_v[row, pl.ds(0, LANES)] = _dot_u(
                    ring.at[par], row, e * stride_e, real_e)

        fire(0, 0)
        fire(1, 1)

        @pl.loop(0, nch - 2, step=2)
        def _steady(c0):
            for par in (0, 1):
                c = c0 + par
                wait(c, par)
                compute(c, par)
                fire(c + 2, par)

        for par in (0, 1):
            c = nch - 2 + par
            wait(c, par)
            compute(c, par)

    _pipelined_pass(idx_p, PP, P, EP, s_v)
    _pipelined_pass(idx_n, NN, N, EN, n_v)

    pltpu.sync_copy(s_v, sc_out.at[pl.ds(base_b, BPW)])
    pltpu.sync_copy(n_v, nsc_out.at[pl.ds(base_b, BPW)])


_sc_scores = pl.kernel(
    _sc_body,
    out_type=(jax.ShapeDtypeStruct((B, LANES), jnp.float32),
              jax.ShapeDtypeStruct((B, LANES), jnp.float32)),
    mesh=plsc.VectorSubcoreMesh(core_axis_name="c", subcore_axis_name="s",
                                num_cores=NC, num_subcores=NS),
    scratch_types=[
        pltpu.VMEM((BPW,), jnp.int32),
        pltpu.VMEM((BPW * PP,), jnp.int32),
        pltpu.VMEM((BPW * NN,), jnp.int32),
        pltpu.VMEM((BPW, D), jnp.float32),
        pltpu.VMEM((2, EN * NN, D), jnp.float32),
        pltpu.VMEM((BPW, LANES), jnp.float32),
        pltpu.VMEM((BPW, LANES), jnp.float32),
        pltpu.SemaphoreType.DMA,
        pltpu.SemaphoreType.DMA,
        pltpu.SemaphoreType.DMA,
    ],
    compiler_params=pltpu.CompilerParams(use_tc_tiling_on_sc=False),
)


def _loss_body(s_ref, n_ref, o_ref):
    s = jnp.sum(s_ref[...], axis=1) * (1.0 / P)
    n = jnp.sum(n_ref[...], axis=1) * (-1.0 / N)
    ls = jnp.minimum(s, 0.0) - jnp.log(1.0 + jnp.exp(-jnp.abs(s)))
    ln = jnp.minimum(n, 0.0) - jnp.log(1.0 + jnp.exp(-jnp.abs(n)))
    o_ref[0, 0] = -(jnp.sum(ls) + jnp.sum(ln)) / B


_loss = pl.pallas_call(
    _loss_body,
    out_shape=jax.ShapeDtypeStruct((1, 1), jnp.float32),
    in_specs=[pl.BlockSpec(memory_space=pltpu.VMEM),
              pl.BlockSpec(memory_space=pltpu.VMEM)],
    out_specs=pl.BlockSpec(memory_space=pltpu.SMEM),
)


def kernel(table, u_pos, v_pos, v_neg):
    p_lin, n_lin = _prep(v_pos, v_neg)
    scores, neg_scores = _sc_scores(table, u_pos, p_lin, n_lin)
    return _loss(scores, neg_scores)[0, 0]


# speedup vs baseline: 6.7195x; 6.7195x over previous
import jax
import jax.numpy as jnp
from jax import lax
from jax.experimental import pallas as pl
from jax.experimental.pallas import tpu as pltpu
from jax.experimental.pallas import tpu_sc as plsc

D = 64
P = 20
N = 50
B = 4096
NC, NS = 2, 16
NW = NC * NS
BPW = B // NW
E = 16
NCH = BPW // E
LANES = 16
KD = D // LANES


def _sc_body(table, u_idx, p_idx, n_idx, sc_out, nsc_out,
             idx_u, idx_p, idx_n, u_rows, ring, s_v, n_v,
             sem_u, sem_g0, sem_g1):
    wid = lax.axis_index("s") * NC + lax.axis_index("c")
    base_b = wid * BPW
    pltpu.sync_copy(u_idx.at[pl.ds(base_b, BPW)], idx_u)
    pltpu.sync_copy(p_idx.at[pl.ds(base_b, BPW)], idx_p)
    pltpu.sync_copy(n_idx.at[pl.ds(base_b, BPW)], idx_n)
    pltpu.async_copy(table.at[idx_u], u_rows, sem_u).wait()
    sems = (sem_g0, sem_g1)

    def _dot_u(buf, row, r0, stride):
        acc = [jnp.zeros((LANES,), jnp.float32) for _ in range(KD)]
        for j in range(stride):
            r = r0 + j
            for k in range(KD):
                acc[k] = acc[k] + buf[r, pl.ds(LANES * k, LANES)]
        dp = acc[0] * u_rows[row, pl.ds(0, LANES)]
        for k in range(1, KD):
            dp = dp + acc[k] * u_rows[row, pl.ds(LANES * k, LANES)]
        return dp

    def _pipelined_pass(fire, wait, rows_per_e, out_v):
        def compute(c, par):
            @pl.loop(0, E, unroll=2)
            def _elem(e):
                row = c * E + e
                out_v[row, pl.ds(0, LANES)] = _dot_u(
                    ring.at[par], row, e * rows_per_e, rows_per_e)

        fire(0, 0)
        fire(1, 1)

        @pl.loop(0, NCH - 2, step=2)
        def _steady(c0):
            for par in (0, 1):
                c = c0 + par
                wait(c, par)
                compute(c, par)
                fire(c + 2, par)

        for par in (0, 1):
            c = NCH - 2 + par
            wait(c, par)
            compute(c, par)

    def fire_p(c, par):
        for j in range(E):
            pltpu.async_copy(table.at[idx_p.at[c * E + j]],
                             ring.at[par, pl.ds(P * j, P)], sems[par])

    def wait_p(c, par):
        for j in range(E):
            pltpu.make_async_copy(table.at[idx_p.at[c * E + j]],
                                  ring.at[par, pl.ds(P * j, P)],
                                  sems[par]).wait()

    _pipelined_pass(fire_p, wait_p, P, s_v)

    def fire_n(c, par):
        for j in range(E):
            pltpu.async_copy(table.at[idx_n.at[c * E + j]],
                             ring.at[par, pl.ds(N * j, N)], sems[par])

    def wait_n(c, par):
        for j in range(E):
            pltpu.make_async_copy(table.at[idx_n.at[c * E + j]],
                                  ring.at[par, pl.ds(N * j, N)],
                                  sems[par]).wait()

    _pipelined_pass(fire_n, wait_n, N, n_v)

    pltpu.sync_copy(s_v, sc_out.at[pl.ds(base_b, BPW)])
    pltpu.sync_copy(n_v, nsc_out.at[pl.ds(base_b, BPW)])


_sc_scores = pl.kernel(
    _sc_body,
    out_type=(jax.ShapeDtypeStruct((B, LANES), jnp.float32),
              jax.ShapeDtypeStruct((B, LANES), jnp.float32)),
    mesh=plsc.VectorSubcoreMesh(core_axis_name="c", subcore_axis_name="s",
                                num_cores=NC, num_subcores=NS),
    scratch_types=[
        pltpu.VMEM((BPW,), jnp.int32),
        pltpu.VMEM((BPW, P), jnp.int32),
        pltpu.VMEM((BPW, N), jnp.int32),
        pltpu.VMEM((BPW, D), jnp.float32),
        pltpu.VMEM((2, E * N, D), jnp.float32),
        pltpu.VMEM((BPW, LANES), jnp.float32),
        pltpu.VMEM((BPW, LANES), jnp.float32),
        pltpu.SemaphoreType.DMA,
        pltpu.SemaphoreType.DMA,
        pltpu.SemaphoreType.DMA,
    ],
    compiler_params=pltpu.CompilerParams(use_tc_tiling_on_sc=False),
)


def _loss_body(s_ref, n_ref, o_ref):
    s = jnp.sum(s_ref[...], axis=1) * (1.0 / P)
    n = jnp.sum(n_ref[...], axis=1) * (-1.0 / N)
    ls = jnp.minimum(s, 0.0) - jnp.log(1.0 + jnp.exp(-jnp.abs(s)))
    ln = jnp.minimum(n, 0.0) - jnp.log(1.0 + jnp.exp(-jnp.abs(n)))
    o_ref[0, 0] = -(jnp.sum(ls) + jnp.sum(ln)) / B


_loss = pl.pallas_call(
    _loss_body,
    out_shape=jax.ShapeDtypeStruct((1, 1), jnp.float32),
    in_specs=[pl.BlockSpec(memory_space=pltpu.VMEM),
              pl.BlockSpec(memory_space=pltpu.VMEM)],
    out_specs=pl.BlockSpec(memory_space=pltpu.SMEM),
)


def kernel(table, u_pos, v_pos, v_neg):
    scores, neg_scores = _sc_scores(table, u_pos, v_pos, v_neg)
    return _loss(scores, neg_scores)[0, 0]


# speedup vs baseline: 7.0955x; 1.0559x over previous
import jax
import jax.numpy as jnp
from jax import lax
from jax.experimental import pallas as pl
from jax.experimental.pallas import tpu as pltpu
from jax.experimental.pallas import tpu_sc as plsc

D = 64
P = 20
N = 50
B = 4096
NC, NS = 2, 16
NW = NC * NS
BPW = B // NW
E = 16
NCH = BPW // E
NBUF = 2
LANES = 16
KD = D // LANES


def _sc_body(table, u_idx, p_idx, n_idx, sc_out, nsc_out,
             idx_u, idx_p, idx_n, u_rows, ring, s_v, n_v,
             sem_u, sem_g0, sem_g1):
    wid = lax.axis_index("s") * NC + lax.axis_index("c")
    base_b = wid * BPW
    pltpu.sync_copy(u_idx.at[pl.ds(base_b, BPW)], idx_u)
    pltpu.sync_copy(p_idx.at[pl.ds(base_b, BPW)], idx_p)
    pltpu.sync_copy(n_idx.at[pl.ds(base_b, BPW)], idx_n)
    pltpu.async_copy(table.at[idx_u], u_rows, sem_u).wait()
    sems = (sem_g0, sem_g1)

    def _dot_u(buf, row, r0, stride):
        acc = [jnp.zeros((LANES,), jnp.float32) for _ in range(KD)]
        for j in range(stride):
            r = r0 + j
            for k in range(KD):
                acc[k] = acc[k] + buf[r, pl.ds(LANES * k, LANES)]
        dp = acc[0] * u_rows[row, pl.ds(0, LANES)]
        for k in range(1, KD):
            dp = dp + acc[k] * u_rows[row, pl.ds(LANES * k, LANES)]
        return dp

    def _pipelined_pass(fire, wait, rows_per_e, out_v):
        def compute(c, par):
            @pl.loop(0, E, unroll=2)
            def _elem(e):
                row = c * E + e
                out_v[row // 8, pl.ds(LANES * (row % 8), LANES)] = _dot_u(
                    ring.at[par], row, e * rows_per_e, rows_per_e)

        for par in range(NBUF):
            fire(par, par)

        @pl.loop(0, NCH - NBUF, step=NBUF)
        def _steady(c0):
            for par in range(NBUF):
                c = c0 + par
                wait(c, par)
                compute(c, par)
                fire(c + NBUF, par)

        for par in range(NBUF):
            c = NCH - NBUF + par
            wait(c, par)
            compute(c, par)

    def fire_p(c, par):
        for j in range(E):
            pltpu.async_copy(table.at[idx_p.at[c * E + j]],
                             ring.at[par, pl.ds(P * j, P)], sems[par])

    def wait_p(c, par):
        for j in range(E):
            pltpu.make_async_copy(table.at[idx_p.at[c * E + j]],
                                  ring.at[par, pl.ds(P * j, P)],
                                  sems[par]).wait()

    _pipelined_pass(fire_p, wait_p, P, s_v)

    def fire_n(c, par):
        for j in range(E):
            pltpu.async_copy(table.at[idx_n.at[c * E + j]],
                             ring.at[par, pl.ds(N * j, N)], sems[par])

    def wait_n(c, par):
        for j in range(E):
            pltpu.make_async_copy(table.at[idx_n.at[c * E + j]],
                                  ring.at[par, pl.ds(N * j, N)],
                                  sems[par]).wait()

    _pipelined_pass(fire_n, wait_n, N, n_v)

    pltpu.sync_copy(s_v, sc_out.at[pl.ds(wid * (BPW // 8), BPW // 8)])
    pltpu.sync_copy(n_v, nsc_out.at[pl.ds(wid * (BPW // 8), BPW // 8)])


_sc_scores = pl.kernel(
    _sc_body,
    out_type=(jax.ShapeDtypeStruct((B // 8, 128), jnp.float32),
              jax.ShapeDtypeStruct((B // 8, 128), jnp.float32)),
    mesh=plsc.VectorSubcoreMesh(core_axis_name="c", subcore_axis_name="s",
                                num_cores=NC, num_subcores=NS),
    scratch_types=[
        pltpu.VMEM((BPW,), jnp.int32),
        pltpu.VMEM((BPW, P), jnp.int32),
        pltpu.VMEM((BPW, N), jnp.int32),
        pltpu.VMEM((BPW, D), jnp.float32),
        pltpu.VMEM((NBUF, E * N, D), jnp.float32),
        pltpu.VMEM((BPW // 8, 128), jnp.float32),
        pltpu.VMEM((BPW // 8, 128), jnp.float32),
        pltpu.SemaphoreType.DMA,
        pltpu.SemaphoreType.DMA,
        pltpu.SemaphoreType.DMA,
    ],
    compiler_params=pltpu.CompilerParams(use_tc_tiling_on_sc=False),
)


def _loss_body(s_ref, n_ref, o_ref):
    grp = (lax.broadcasted_iota(jnp.int32, (128, 8), 0) // LANES
           == lax.broadcasted_iota(jnp.int32, (128, 8), 1))
    m = grp.astype(jnp.float32)
    s = jnp.dot(s_ref[...], m) * (1.0 / P)
    n = jnp.dot(n_ref[...], m) * (-1.0 / N)
    ls = jnp.minimum(s, 0.0) - jnp.log(1.0 + jnp.exp(-jnp.abs(s)))
    ln = jnp.minimum(n, 0.0) - jnp.log(1.0 + jnp.exp(-jnp.abs(n)))
    o_ref[0, 0] = -(jnp.sum(ls) + jnp.sum(ln)) / B


_loss = pl.pallas_call(
    _loss_body,
    out_shape=jax.ShapeDtypeStruct((1, 1), jnp.float32),
    cost_estimate=pl.CostEstimate(flops=B * 16 * 2, bytes_accessed=B * 128,
                                  transcendentals=2 * B),
    in_specs=[pl.BlockSpec(memory_space=pltpu.VMEM),
              pl.BlockSpec(memory_space=pltpu.VMEM)],
    out_specs=pl.BlockSpec(memory_space=pltpu.SMEM),
)


def kernel(table, u_pos, v_pos, v_neg):
    scores, neg_scores = _sc_scores(table, u_pos, v_pos, v_neg)
    return _loss(scores, neg_scores)[0, 0]
